# baseline (speedup 1.0000x reference)
import functools
import math

import jax
import jax.numpy as jnp
from jax import lax
from jax.experimental import pallas as pl
from jax.experimental.pallas import tpu as pltpu
from jax.experimental.pallas import tpu_sc as plsc

NUM_ROWS = 4096 * 200
DIM = 64
LANES = 16
NCORES = 2
NSUB = 16
NW = NCORES * NSUB
PER_W = NUM_ROWS // NW
CHUNK = 128
NCHUNK = PER_W // CHUNK
NBUF = 4
SCALE = math.sqrt(DIM)


def _emb_body(idx_hbm, table_hbm, out_hbm, idx_v, *scratch):
    grows = scratch[0:NBUF]
    srows = scratch[NBUF:2 * NBUF]
    gsems = scratch[2 * NBUF:3 * NBUF]
    osems = scratch[3 * NBUF:4 * NBUF]

    wid = lax.axis_index("s") * NCORES + lax.axis_index("c")
    base = wid * PER_W

    pltpu.sync_copy(idx_hbm.at[wid], idx_v)

    def start_gather(g, b):
        pltpu.async_copy(table_hbm.at[idx_v.at[g]], grows[b], gsems[b])

    def wait_gather(g, b):
        pltpu.make_async_copy(table_hbm.at[idx_v.at[g]], grows[b], gsems[b]).wait()

    def out_slice(g):
        return out_hbm.at[pl.ds(base + g * CHUNK, CHUNK)]

    def start_store(g, b):
        pltpu.async_copy(srows[b], out_slice(g), osems[b])

    def wait_store(g, b):
        pltpu.make_async_copy(srows[b], out_slice(g), osems[b]).wait()

    def scale(b):
        def row(r, carry):
            for c in range(DIM // LANES):
                sl = pl.ds(c * LANES, LANES)
                srows[b][r, sl] = grows[b][r, sl] * SCALE
            return carry
        lax.fori_loop(0, CHUNK, row, 0, unroll=4)

    for b in range(NBUF):
        start_gather(b, b)
    for b in range(NBUF):
        wait_gather(b, b)
        scale(b)
        start_store(b, b)
        start_gather(b + NBUF, b)

    def group(k, carry):
        g0 = k * NBUF
        for b in range(NBUF):
            g = g0 + b
            wait_gather(g, b)
            wait_store(g - NBUF, b)
            scale(b)
            start_store(g, b)
            start_gather(g + NBUF, b)
        return carry
    lax.fori_loop(1, NCHUNK // NBUF - 1, group, 0)

    for b in range(NBUF):
        g = NCHUNK - NBUF + b
        wait_gather(g, b)
        wait_store(g - NBUF, b)
        scale(b)
        start_store(g, b)
    for b in range(NBUF):
        wait_store(NCHUNK - NBUF + b, b)


_emb = functools.partial(
    pl.kernel,
    mesh=plsc.VectorSubcoreMesh(core_axis_name="c", subcore_axis_name="s"),
    out_type=jax.ShapeDtypeStruct((NUM_ROWS, DIM), jnp.float32),
    compiler_params=pltpu.CompilerParams(use_tc_tiling_on_sc=False),
    scratch_types=(
        [pltpu.VMEM((NCHUNK, CHUNK), jnp.int32)]
        + [pltpu.VMEM((CHUNK, DIM), jnp.float32)] * (2 * NBUF)
        + [pltpu.SemaphoreType.DMA] * (2 * NBUF)
    ),
)(_emb_body)


def kernel(x, table):
    xf = x.reshape(NW, NCHUNK, CHUNK).astype(jnp.int32)
    out = _emb(xf, table)
    return out.reshape(x.shape[0], x.shape[1], DIM)
